# Initial kernel scaffold; baseline (speedup 1.0000x reference)
import jax
import jax.numpy as jnp
from jax.experimental import pallas as pl


def kernel(in_features, pair_first, pair_second, dist_pairs, mu, sigma, int_weights, self_W, self_b):
    raise NotImplementedError("write your pallas kernel here")


# speedup vs baseline: 7.3275x; 7.3275x over previous
import functools

import jax
import jax.numpy as jnp
from jax import lax
from jax.experimental import pallas as pl
from jax.experimental.pallas import tpu as pltpu
from jax.experimental.pallas import tpu_sc as plsc

N = 10000
E = 320000
NF = 128
ND = 20
HARD_CUTOFF = 6.5

NW = 32
CHUNK = 128
NCHUNKS = E // CHUNK
STEPS = (NCHUNKS + NW - 1) // NW

BB = 512
NB = E // BB

NFULL = N // CHUNK
NREM = N - NFULL * CHUNK


def _sc_gather_body(x_hbm, ps_hbm, g_hbm, idx_v, rows_v, sem):
    wid = lax.axis_index("c") * 16 + lax.axis_index("s")

    def step(j, carry):
        c = j * NW + wid

        @pl.when(c < NCHUNKS)
        def _():
            base = c * CHUNK
            pltpu.sync_copy(ps_hbm.at[pl.ds(base, CHUNK)], idx_v)
            pltpu.async_copy(x_hbm.at[idx_v], rows_v, sem).wait()
            pltpu.sync_copy(rows_v, g_hbm.at[pl.ds(base, CHUNK)])

        return carry

    lax.fori_loop(0, STEPS, step, 0)


def _sc_scatter_body(z_hbm, pf_hbm, out_hbm, idx_v, rows_v, acc_sh):
    cid = lax.axis_index("c")
    sid = lax.axis_index("s")
    wid = cid * 16 + sid

    zeros16 = jnp.zeros((16,), jnp.float32)

    def zstep(i, carry):
        r = i // (NF // 16)
        col = (i % (NF // 16)) * 16
        rows_v[r, pl.ds(col, 16)] = zeros16
        return carry

    lax.fori_loop(0, CHUNK * (NF // 16), zstep, 0)

    for i in range((NFULL + 15) // 16):
        blk = sid + i * 16

        @pl.when(blk < NFULL)
        def _():
            pltpu.sync_copy(rows_v, acc_sh.at[pl.ds(blk * CHUNK, CHUNK)])

    @pl.when(sid == 0)
    def _():
        pltpu.sync_copy(rows_v.at[pl.ds(0, NREM)],
                        acc_sh.at[pl.ds(NFULL * CHUNK, NREM)])

    plsc.subcore_barrier()

    def step(j, carry):
        c = j * NW + wid

        @pl.when(c < NCHUNKS)
        def _():
            base = c * CHUNK
            pltpu.sync_copy(pf_hbm.at[pl.ds(base, CHUNK)], idx_v)
            pltpu.sync_copy(z_hbm.at[pl.ds(base, CHUNK)], rows_v)
            pltpu.sync_copy(rows_v, acc_sh.at[idx_v], add=True)

        return carry

    lax.fori_loop(0, STEPS, step, 0)
    plsc.subcore_barrier()

    for i in range((NFULL + 15) // 16):
        blk = sid + i * 16

        @pl.when(blk < NFULL)
        def _():
            pltpu.sync_copy(acc_sh.at[pl.ds(blk * CHUNK, CHUNK)], rows_v)
            pltpu.sync_copy(rows_v, out_hbm.at[pl.ds(cid * N + blk * CHUNK, CHUNK)])

    @pl.when(sid == 0)
    def _():
        pltpu.sync_copy(acc_sh.at[pl.ds(NFULL * CHUNK, NREM)],
                        rows_v.at[pl.ds(0, NREM)])
        pltpu.sync_copy(rows_v.at[pl.ds(0, NREM)],
                        out_hbm.at[pl.ds(cid * N + NFULL * CHUNK, NREM)])


def _tc_z_body(g_ref, d_ref, w_ref, mu_ref, sg_ref, z_ref):
    g = g_ref[...]
    d = d_ref[0]
    inv = 1.0 / d
    cut = jnp.where(
        d < HARD_CUTOFF,
        jnp.cos(d * (jnp.pi / (2.0 * HARD_CUTOFF))) ** 2,
        0.0,
    )
    h = jnp.dot(g, w_ref[...], preferred_element_type=jnp.float32)
    acc = jnp.zeros((BB, NF), jnp.float32)
    for k in range(ND):
        t = (inv - mu_ref[0, k]) / sg_ref[0, k]
        s_k = jnp.exp(-0.5 * t * t) * cut
        acc = acc + s_k * h[:, k * NF:(k + 1) * NF]
    z_ref[...] = acc


TD = 400


def _tc_out_body(p_ref, x_ref, w_ref, b_ref, o_ref):
    s = jnp.dot(x_ref[...], w_ref[...], preferred_element_type=jnp.float32)
    o_ref[...] = p_ref[0] + p_ref[1] + s + b_ref[...]


def kernel(in_features, pair_first, pair_second, dist_pairs, mu, sigma,
           int_weights, self_W, self_b):
    ps = pair_second.astype(jnp.int32)
    pf = pair_first.astype(jnp.int32)
    x = in_features.astype(jnp.float32)

    mesh = plsc.VectorSubcoreMesh(core_axis_name="c", subcore_axis_name="s")

    gather = pl.kernel(
        _sc_gather_body,
        out_type=jax.ShapeDtypeStruct((E, NF), jnp.float32),
        mesh=mesh,
        scratch_types=[
            pltpu.VMEM((CHUNK,), jnp.int32),
            pltpu.VMEM((CHUNK, NF), jnp.float32),
            pltpu.SemaphoreType.DMA,
        ],
    )
    g = gather(x, ps)

    wm = jnp.transpose(int_weights, (2, 0, 1)).reshape(NF, ND * NF)
    dist4 = dist_pairs.astype(jnp.float32).reshape(NB, BB, 1)
    mu2 = mu.astype(jnp.float32).reshape(1, ND)
    sg2 = sigma.astype(jnp.float32).reshape(1, ND)
    z = pl.pallas_call(
        _tc_z_body,
        grid=(NB,),
        in_specs=[
            pl.BlockSpec((BB, NF), lambda b: (b, 0)),
            pl.BlockSpec((1, BB, 1), lambda b: (b, 0, 0)),
            pl.BlockSpec((NF, ND * NF), lambda b: (0, 0)),
            pl.BlockSpec(memory_space=pltpu.SMEM),
            pl.BlockSpec(memory_space=pltpu.SMEM),
        ],
        out_specs=pl.BlockSpec((BB, NF), lambda b: (b, 0)),
        out_shape=jax.ShapeDtypeStruct((E, NF), jnp.float32),
    )(g, dist4, wm, mu2, sg2)

    scatter = pl.kernel(
        _sc_scatter_body,
        out_type=jax.ShapeDtypeStruct((2 * N, NF), jnp.float32),
        mesh=mesh,
        scratch_types=[
            pltpu.VMEM((CHUNK,), jnp.int32),
            pltpu.VMEM((CHUNK, NF), jnp.float32),
            pltpu.VMEM_SHARED((N, NF), jnp.float32),
        ],
    )
    partial = scatter(z, pf).reshape(2, N, NF)

    swt = jnp.transpose(self_W, (1, 0)).astype(jnp.float32)
    b2 = self_b.astype(jnp.float32).reshape(1, NF)
    out = pl.pallas_call(
        _tc_out_body,
        grid=(N // TD,),
        in_specs=[
            pl.BlockSpec((2, TD, NF), lambda b: (0, b, 0)),
            pl.BlockSpec((TD, NF), lambda b: (b, 0)),
            pl.BlockSpec((NF, NF), lambda b: (0, 0)),
            pl.BlockSpec((1, NF), lambda b: (0, 0)),
        ],
        out_specs=pl.BlockSpec((TD, NF), lambda b: (b, 0)),
        out_shape=jax.ShapeDtypeStruct((N, NF), jnp.float32),
    )(partial, x, swt, b2)
    return out
